# baseline (speedup 1.0000x reference)
import functools

import jax
import jax.numpy as jnp
from jax import lax
from jax.experimental import pallas as pl
from jax.experimental.pallas import tpu as pltpu
from jax.experimental.pallas import tpu_sc as plsc


@functools.cache
def _make_gather(V, D, B):
  info = plsc.get_sparse_core_info()
  NC, NS = info.num_cores, info.num_subcores
  NW = NC * NS
  assert B % (8 * NW) == 0 and D % info.num_lanes == 0
  b_per_w = B // NW
  chunk = min(128, b_per_w)
  n_chunks = b_per_w // chunk
  mesh = plsc.VectorSubcoreMesh(core_axis_name="c", subcore_axis_name="s")

  @functools.partial(
      pl.kernel,
      mesh=mesh,
      compiler_params=pltpu.CompilerParams(use_tc_tiling_on_sc=False),
      out_type=jax.ShapeDtypeStruct((B, D), jnp.float32),
      scratch_types=[
          pltpu.VMEM((b_per_w,), jnp.int32),
          pltpu.VMEM((b_per_w, D), jnp.float32),
          pltpu.SemaphoreType.DMA,
      ],
  )
  def gather_kernel(idx_hbm, table_hbm, out_hbm, idx_v, rows_v, sem):
    wid = lax.axis_index("s") * NC + lax.axis_index("c")
    base = wid * b_per_w
    pltpu.sync_copy(idx_hbm.at[pl.ds(base, b_per_w)], idx_v)
    copies = []
    for j in range(n_chunks):
      copies.append(
          pltpu.async_copy(
              table_hbm.at[idx_v.at[pl.ds(j * chunk, chunk)]],
              rows_v.at[pl.ds(j * chunk, chunk)],
              sem,
          )
      )
    for c in copies:
      c.wait()
    pltpu.sync_copy(rows_v, out_hbm.at[pl.ds(base, b_per_w)])

  return gather_kernel


def kernel(speaker_ids, embedding_table):
  B, = speaker_ids.shape
  V, D = embedding_table.shape
  ids = speaker_ids.astype(jnp.int32)
  return _make_gather(V, D, B)(ids, embedding_table)


# speedup vs baseline: 1.9216x; 1.9216x over previous
import functools

import jax
import jax.numpy as jnp
from jax import lax
from jax.experimental import pallas as pl
from jax.experimental.pallas import tpu as pltpu
from jax.experimental.pallas import tpu_sc as plsc


@functools.cache
def _make_gather_t(V, D, B):
  info = plsc.get_sparse_core_info()
  NC, NS, L = info.num_cores, info.num_subcores, info.num_lanes
  NW = NC * NS
  assert D % NW == 0 and B % L == 0
  rows_per_w = D // NW
  CH = min(4096, B)
  mesh = plsc.VectorSubcoreMesh(core_axis_name="c", subcore_axis_name="s")

  @functools.partial(
      pl.kernel,
      mesh=mesh,
      compiler_params=pltpu.CompilerParams(
          use_tc_tiling_on_sc=True, needs_layout_passes=False
      ),
      out_type=jax.ShapeDtypeStruct((D, B), jnp.float32),
      scratch_types=[
          pltpu.VMEM((V,), jnp.float32),
          pltpu.VMEM((B,), jnp.int32),
          pltpu.VMEM((CH,), jnp.float32),
      ],
  )
  def gather_kernel(ids_hbm, tt_hbm, out_hbm, row_v, ids_v, out_v):
    wid = lax.axis_index("s") * NC + lax.axis_index("c")
    pltpu.sync_copy(ids_hbm, ids_v)
    for i in range(rows_per_w):
      d = wid * rows_per_w + i
      pltpu.sync_copy(tt_hbm.at[d], row_v)

      for cb in range(B // CH):
        def body(j, _):
          j16 = cb * CH + j * L
          idx = ids_v[pl.ds(j16, L)]
          out_v[pl.ds(j * L, L)] = plsc.load_gather(row_v, [idx])
          return 0

        lax.fori_loop(0, CH // L, body, 0)
        pltpu.sync_copy(out_v, out_hbm.at[d, pl.ds(cb * CH, CH)])

  return gather_kernel


def kernel(speaker_ids, embedding_table):
  B, = speaker_ids.shape
  V, D = embedding_table.shape
  ids = speaker_ids.astype(jnp.int32)
  out_t = _make_gather_t(V, D, B)(ids, embedding_table.T)
  return out_t.T
